# baseline (speedup 1.0000x reference)
import functools

import jax
import jax.numpy as jnp
from jax import lax
from jax.experimental import pallas as pl
from jax.experimental.pallas import tpu as pltpu
from jax.experimental.pallas import tpu_sc as plsc

NC = 2
NS = 16
NW = NC * NS
B = 4
NF = 4096
NCRS = 1024
NG = 4096
SLOTS = NW // B

L = 16
QB = 4
U = 4
BIG = 1e30

RT = 256
CT = 512
NRT = NF // RT
NCT = NG // CT


def _norms(xr, yr, zr, nr, n):
    def body(i, carry):
        off = i * L
        x = xr[pl.ds(off, L)]
        y = yr[pl.ds(off, L)]
        z = zr[pl.ds(off, L)]
        nr[pl.ds(off, L)] = x * x + y * y + z * z
        return carry
    lax.fori_loop(0, n // L, body, 0)


def _fill(ref, n, val):
    v = jnp.full((L,), val, jnp.float32)
    def body(i, carry):
        ref[pl.ds(i * L, L)] = v
        return carry
    lax.fori_loop(0, n // L, body, 0)


def _fused_pass(qx, qy, qz, qn, q0, nq, tx, ty, tz, tn, nt, cm):
    def qgroup(qg, acc):
        qbase = q0 + qg * L
        gqx = qx[pl.ds(qbase, L)]
        gqy = qy[pl.ds(qbase, L)]
        gqz = qz[pl.ds(qbase, L)]
        gqn = qn[pl.ds(qbase, L)]

        for sb in range(L // QB):
            qm2 = []
            qnb = []
            for q in range(QB):
                i = sb * QB + q
                qm2.append((jnp.full((L,), gqx[i]) * -2.0,
                            jnp.full((L,), gqy[i]) * -2.0,
                            jnp.full((L,), gqz[i]) * -2.0))
                qnb.append(jnp.full((L,), gqn[i]))

            def tchunk(tc, carry):
                ms = list(carry)
                base = tc * (U * L)
                for u in range(U):
                    off = base + u * L
                    vx = tx[pl.ds(off, L)]
                    vy = ty[pl.ds(off, L)]
                    vz = tz[pl.ds(off, L)]
                    vn = tn[pl.ds(off, L)]
                    ds = []
                    for q in range(QB):
                        ax, ay, az = qm2[q]
                        d = vn + vx * ax + vy * ay + vz * az
                        ms[q] = jnp.minimum(ms[q], d)
                        ds.append(d + qnb[q])
                    e = jnp.minimum(jnp.minimum(ds[0], ds[1]),
                                    jnp.minimum(ds[2], ds[3]))
                    cm[pl.ds(off, L)] = jnp.minimum(cm[pl.ds(off, L)], e)
                return tuple(ms)

            init = tuple(jnp.full((L,), BIG, jnp.float32)
                         for _ in range(QB))
            ms = lax.fori_loop(0, nt // (U * L), tchunk, init)
            for q in range(QB):
                acc = acc + jnp.min(ms[q]) + gqn[sb * QB + q]
        return acc

    return lax.fori_loop(0, nq // L, qgroup, jnp.float32(0.0))


def _merge_cols(shared, row0, slot, mb, n_per_slot):
    col0 = slot * n_per_slot
    for r in range(SLOTS):
        pltpu.sync_copy(shared.at[row0 + r, pl.ds(col0, n_per_slot)],
                        mb.at[r])

    def chunk(c, acc):
        off = c * L
        m = mb[0, pl.ds(off, L)]
        for r in range(1, SLOTS):
            m = jnp.minimum(m, mb[r, pl.ds(off, L)])
        return acc + m

    sv = lax.fori_loop(0, n_per_slot // L, chunk,
                       jnp.zeros((L,), jnp.float32))
    return jnp.sum(sv)


def _sc_body(gt_hbm, crs_hbm, out_hbm,
             gx, gy, gz, cx, cy, cz, gn, cn,
             cmc, mbc, ob, shc, sem):
    core = lax.axis_index("c")
    sub = lax.axis_index("s")
    wid = core * NS + sub
    b = wid // SLOTS
    slot = wid % SLOTS
    row0 = (sub // SLOTS) * SLOTS

    copies = [
        pltpu.async_copy(gt_hbm.at[b * 3 + 0], gx, sem),
        pltpu.async_copy(gt_hbm.at[b * 3 + 1], gy, sem),
        pltpu.async_copy(gt_hbm.at[b * 3 + 2], gz, sem),
        pltpu.async_copy(crs_hbm.at[b * 3 + 0], cx, sem),
        pltpu.async_copy(crs_hbm.at[b * 3 + 1], cy, sem),
        pltpu.async_copy(crs_hbm.at[b * 3 + 2], cz, sem),
    ]
    for c in copies:
        c.wait()

    _norms(gx, gy, gz, gn, NG)
    _norms(cx, cy, cz, cn, NCRS)
    _fill(cmc, NG, BIG)

    nc_s = NCRS // SLOTS
    s_cg = _fused_pass(cx, cy, cz, cn, slot * nc_s, nc_s,
                       gx, gy, gz, gn, NG, cmc)

    pltpu.sync_copy(cmc, shc.at[sub])
    plsc.subcore_barrier()

    ng_s = NG // SLOTS
    s_gc = _merge_cols(shc, row0, slot, mbc, ng_s)

    lane = lax.iota(jnp.int32, L)
    v = jnp.where(lane == 0, s_cg, jnp.where(lane == 1, s_gc, 0.0))
    ob[...] = v
    pltpu.sync_copy(ob, out_hbm.at[wid])


def _tc_body(x_ref, g_ref, out_ref, racc, cacc):
    b = pl.program_id(0)
    rt = pl.program_id(1)
    ct = pl.program_id(2)

    xb = x_ref[0]
    gb = g_ref[0]

    xn = jnp.sum(xb * xb, axis=0, keepdims=True)
    gn = jnp.sum(gb * gb, axis=0, keepdims=True)
    ones_x = jnp.ones((1, RT), jnp.float32)
    ones_g = jnp.ones((1, CT), jnp.float32)
    x5 = jnp.concatenate([xb * -2.0, xn, ones_x], axis=0)
    g5 = jnp.concatenate([gb, ones_g, gn], axis=0)

    d2 = lax.dot_general(x5, g5, (((0,), (0,)), ((), ())),
                         precision=lax.Precision.HIGHEST,
                         preferred_element_type=jnp.float32)

    @pl.when(ct == 0)
    def _():
        racc[...] = d2

    @pl.when(ct != 0)
    def _():
        racc[...] = jnp.minimum(racc[...], d2)

    @pl.when(rt == 0)
    def _():
        cacc[ct] = d2

    @pl.when(rt != 0)
    def _():
        cacc[ct] = jnp.minimum(cacc[ct], d2)

    @pl.when(ct == NCT - 1)
    def _():
        s = jnp.sum(jnp.min(racc[...], axis=1))
        prev = jnp.where(rt == 0, 0.0, out_ref[b, 0])
        out_ref[b, 0] = prev + s

    @pl.when(rt == NRT - 1)
    def _():
        s = jnp.sum(jnp.min(cacc[ct], axis=0))
        prev = jnp.where(ct == 0, 0.0, out_ref[b, 1])
        out_ref[b, 1] = prev + s


def _tc_fine(fine3, gt3):
    return pl.pallas_call(
        _tc_body,
        grid=(B, NRT, NCT),
        in_specs=[
            pl.BlockSpec((1, 3, RT), lambda b, rt, ct: (b, 0, rt)),
            pl.BlockSpec((1, 3, CT), lambda b, rt, ct: (b, 0, ct)),
        ],
        out_specs=pl.BlockSpec(memory_space=pltpu.SMEM),
        out_shape=jax.ShapeDtypeStruct((B, 2), jnp.float32),
        scratch_shapes=[
            pltpu.VMEM((RT, CT), jnp.float32),
            pltpu.VMEM((NCT, RT, CT), jnp.float32),
        ],
    )(fine3, gt3)


@jax.jit
def kernel(coarse, fine, gt, alpha):
    gt2 = gt.reshape(B * 3, NG)
    fine2 = jnp.transpose(fine, (0, 2, 1)).reshape(B * 3, NF)
    crs2 = jnp.transpose(coarse, (0, 2, 1)).reshape(B * 3, NCRS)

    mesh = plsc.VectorSubcoreMesh(core_axis_name="c", subcore_axis_name="s")
    run = functools.partial(
        pl.kernel,
        mesh=mesh,
        compiler_params=pltpu.CompilerParams(needs_layout_passes=False),
        out_type=jax.ShapeDtypeStruct((NW, L), jnp.float32),
        scratch_types=[
            pltpu.VMEM((NG,), jnp.float32),
            pltpu.VMEM((NG,), jnp.float32),
            pltpu.VMEM((NG,), jnp.float32),
            pltpu.VMEM((NCRS,), jnp.float32),
            pltpu.VMEM((NCRS,), jnp.float32),
            pltpu.VMEM((NCRS,), jnp.float32),
            pltpu.VMEM((NG,), jnp.float32),
            pltpu.VMEM((NCRS,), jnp.float32),
            pltpu.VMEM((NG,), jnp.float32),
            pltpu.VMEM((SLOTS, NG // SLOTS), jnp.float32),
            pltpu.VMEM((L,), jnp.float32),
            pltpu.VMEM_SHARED((NS, NG), jnp.float32),
            pltpu.SemaphoreType.DMA,
        ],
    )(_sc_body)
    sc_part = run(gt2, crs2)
    tc_part = _tc_fine(fine2.reshape(B, 3, NF), gt)

    p = sc_part.reshape(B, SLOTS, L)[:, :, :2].sum(axis=1)
    cham_coarse = p[:, 0] / NCRS + p[:, 1] / NG
    cham_fine = tc_part[:, 0] / NF + tc_part[:, 1] / NG
    loss_fine = jnp.mean(cham_fine)
    loss_coarse = jnp.mean(cham_coarse)
    loss = loss_coarse + alpha * loss_fine
    return (loss, loss_coarse, loss_fine)


# speedup vs baseline: 3.1608x; 3.1608x over previous
import functools

import jax
import jax.numpy as jnp
from jax import lax
from jax.experimental import pallas as pl
from jax.experimental.pallas import tpu as pltpu
from jax.experimental.pallas import tpu_sc as plsc

NC = 2
NS = 16
NW = NC * NS
B = 4
NF = 4096
NCRS = 1024
NG = 4096
SLOTS = NW // B

L = 16
QB = 4
U = 4
BIG = 1e30

RT = 256
CW = 128
NRT = NF // RT
NCW = NG // CW


def _norms(xr, yr, zr, nr, n):
    def body(i, carry):
        off = i * L
        x = xr[pl.ds(off, L)]
        y = yr[pl.ds(off, L)]
        z = zr[pl.ds(off, L)]
        nr[pl.ds(off, L)] = x * x + y * y + z * z
        return carry
    lax.fori_loop(0, n // L, body, 0)


def _fill(ref, n, val):
    v = jnp.full((L,), val, jnp.float32)
    def body(i, carry):
        ref[pl.ds(i * L, L)] = v
        return carry
    lax.fori_loop(0, n // L, body, 0)


def _fused_pass(qx, qy, qz, qn, q0, nq, tx, ty, tz, tn, nt, cm):
    def qgroup(qg, acc):
        qbase = q0 + qg * L
        gqx = qx[pl.ds(qbase, L)]
        gqy = qy[pl.ds(qbase, L)]
        gqz = qz[pl.ds(qbase, L)]
        gqn = qn[pl.ds(qbase, L)]

        for sb in range(L // QB):
            qm2 = []
            qnb = []
            for q in range(QB):
                i = sb * QB + q
                qm2.append((jnp.full((L,), gqx[i]) * -2.0,
                            jnp.full((L,), gqy[i]) * -2.0,
                            jnp.full((L,), gqz[i]) * -2.0))
                qnb.append(jnp.full((L,), gqn[i]))

            def tchunk(tc, carry):
                ms = list(carry)
                base = tc * (U * L)
                for u in range(U):
                    off = base + u * L
                    vx = tx[pl.ds(off, L)]
                    vy = ty[pl.ds(off, L)]
                    vz = tz[pl.ds(off, L)]
                    vn = tn[pl.ds(off, L)]
                    ds = []
                    for q in range(QB):
                        ax, ay, az = qm2[q]
                        d = vn + vx * ax + vy * ay + vz * az
                        ms[q] = jnp.minimum(ms[q], d)
                        ds.append(d + qnb[q])
                    e = jnp.minimum(jnp.minimum(ds[0], ds[1]),
                                    jnp.minimum(ds[2], ds[3]))
                    cm[pl.ds(off, L)] = jnp.minimum(cm[pl.ds(off, L)], e)
                return tuple(ms)

            init = tuple(jnp.full((L,), BIG, jnp.float32)
                         for _ in range(QB))
            ms = lax.fori_loop(0, nt // (U * L), tchunk, init)
            for q in range(QB):
                acc = acc + jnp.min(ms[q]) + gqn[sb * QB + q]
        return acc

    return lax.fori_loop(0, nq // L, qgroup, jnp.float32(0.0))


def _merge_cols(shared, row0, slot, mb, n_per_slot):
    col0 = slot * n_per_slot
    for r in range(SLOTS):
        pltpu.sync_copy(shared.at[row0 + r, pl.ds(col0, n_per_slot)],
                        mb.at[r])

    def chunk(c, acc):
        off = c * L
        m = mb[0, pl.ds(off, L)]
        for r in range(1, SLOTS):
            m = jnp.minimum(m, mb[r, pl.ds(off, L)])
        return acc + m

    sv = lax.fori_loop(0, n_per_slot // L, chunk,
                       jnp.zeros((L,), jnp.float32))
    return jnp.sum(sv)


def _sc_body(gt_hbm, crs_hbm, out_hbm,
             gx, gy, gz, cx, cy, cz, gn, cn,
             cmc, mbc, ob, shc, sem):
    core = lax.axis_index("c")
    sub = lax.axis_index("s")
    wid = core * NS + sub
    b = wid // SLOTS
    slot = wid % SLOTS
    row0 = (sub // SLOTS) * SLOTS

    copies = [
        pltpu.async_copy(gt_hbm.at[b * 3 + 0], gx, sem),
        pltpu.async_copy(gt_hbm.at[b * 3 + 1], gy, sem),
        pltpu.async_copy(gt_hbm.at[b * 3 + 2], gz, sem),
        pltpu.async_copy(crs_hbm.at[b * 3 + 0], cx, sem),
        pltpu.async_copy(crs_hbm.at[b * 3 + 1], cy, sem),
        pltpu.async_copy(crs_hbm.at[b * 3 + 2], cz, sem),
    ]
    for c in copies:
        c.wait()

    _norms(gx, gy, gz, gn, NG)
    _norms(cx, cy, cz, cn, NCRS)
    _fill(cmc, NG, BIG)

    nc_s = NCRS // SLOTS
    s_cg = _fused_pass(cx, cy, cz, cn, slot * nc_s, nc_s,
                       gx, gy, gz, gn, NG, cmc)

    pltpu.sync_copy(cmc, shc.at[sub])
    plsc.subcore_barrier()

    ng_s = NG // SLOTS
    s_gc = _merge_cols(shc, row0, slot, mbc, ng_s)

    lane = lax.iota(jnp.int32, L)
    v = jnp.where(lane == 0, s_cg, jnp.where(lane == 1, s_gc, 0.0))
    ob[...] = v
    pltpu.sync_copy(ob, out_hbm.at[wid])


def _mintree(vals):
    while len(vals) > 1:
        nxt = [jnp.minimum(vals[i], vals[i + 1])
               for i in range(0, len(vals) - 1, 2)]
        if len(vals) % 2:
            nxt.append(vals[-1])
        vals = nxt
    return vals[0]


def _tc_body(x_ref, g_ref, out_ref, cacc):
    b = pl.program_id(0)
    rt = pl.program_id(1)

    xb = x_ref[0]
    gb = g_ref[0]

    @pl.when(rt == 0)
    def _():
        cacc[...] = jnp.full((NCW, 8, CW), BIG, jnp.float32)

    bx = jnp.broadcast_to(xb[:, 0:1], (RT, CW))
    by = jnp.broadcast_to(xb[:, 1:2], (RT, CW))
    bz = jnp.broadcast_to(xb[:, 2:3], (RT, CW))

    rowacc = jnp.full((RT, CW), BIG, jnp.float32)
    for cw in range(NCW):
        c0 = cw * CW
        dx = bx - gb[0:1, c0:c0 + CW]
        dy = by - gb[1:2, c0:c0 + CW]
        dz = bz - gb[2:3, c0:c0 + CW]
        d2 = dx * dx + dy * dy + dz * dz
        rowacc = jnp.minimum(rowacc, d2)
        cf = _mintree([d2[r * 8:(r + 1) * 8, :] for r in range(RT // 8)])
        cacc[cw] = jnp.minimum(cacc[cw], cf)

    s_row = jnp.sum(jnp.min(rowacc, axis=1))
    prev = jnp.where(rt == 0, 0.0, out_ref[b, 0])
    out_ref[b, 0] = prev + s_row

    @pl.when(rt == NRT - 1)
    def _():
        sv = jnp.zeros((1, CW), jnp.float32)
        for cw in range(NCW):
            e = cacc[cw]
            m = _mintree([e[r:r + 1, :] for r in range(8)])
            sv = sv + m
        out_ref[b, 1] = jnp.sum(sv)


def _tc_fine(fine3, gt3):
    return pl.pallas_call(
        _tc_body,
        grid=(B, NRT),
        in_specs=[
            pl.BlockSpec((1, RT, 3), lambda b, rt: (b, rt, 0)),
            pl.BlockSpec((1, 3, NG), lambda b, rt: (b, 0, 0)),
        ],
        out_specs=pl.BlockSpec(memory_space=pltpu.SMEM),
        out_shape=jax.ShapeDtypeStruct((B, 2), jnp.float32),
        scratch_shapes=[
            pltpu.VMEM((NCW, 8, CW), jnp.float32),
        ],
    )(fine3, gt3)


@jax.jit
def kernel(coarse, fine, gt, alpha):
    gt2 = gt.reshape(B * 3, NG)
    fine2 = jnp.transpose(fine, (0, 2, 1)).reshape(B * 3, NF)
    crs2 = jnp.transpose(coarse, (0, 2, 1)).reshape(B * 3, NCRS)

    mesh = plsc.VectorSubcoreMesh(core_axis_name="c", subcore_axis_name="s")
    run = functools.partial(
        pl.kernel,
        mesh=mesh,
        compiler_params=pltpu.CompilerParams(needs_layout_passes=False),
        out_type=jax.ShapeDtypeStruct((NW, L), jnp.float32),
        scratch_types=[
            pltpu.VMEM((NG,), jnp.float32),
            pltpu.VMEM((NG,), jnp.float32),
            pltpu.VMEM((NG,), jnp.float32),
            pltpu.VMEM((NCRS,), jnp.float32),
            pltpu.VMEM((NCRS,), jnp.float32),
            pltpu.VMEM((NCRS,), jnp.float32),
            pltpu.VMEM((NG,), jnp.float32),
            pltpu.VMEM((NCRS,), jnp.float32),
            pltpu.VMEM((NG,), jnp.float32),
            pltpu.VMEM((SLOTS, NG // SLOTS), jnp.float32),
            pltpu.VMEM((L,), jnp.float32),
            pltpu.VMEM_SHARED((NS, NG), jnp.float32),
            pltpu.SemaphoreType.DMA,
        ],
    )(_sc_body)
    sc_part = run(gt2, crs2)
    tc_part = _tc_fine(fine, gt)

    p = sc_part.reshape(B, SLOTS, L)[:, :, :2].sum(axis=1)
    cham_coarse = p[:, 0] / NCRS + p[:, 1] / NG
    cham_fine = tc_part[:, 0] / NF + tc_part[:, 1] / NG
    loss_fine = jnp.mean(cham_fine)
    loss_coarse = jnp.mean(cham_coarse)
    loss = loss_coarse + alpha * loss_fine
    return (loss, loss_coarse, loss_fine)


# speedup vs baseline: 3.6481x; 1.1542x over previous
import functools

import jax
import jax.numpy as jnp
from jax import lax
from jax.experimental import pallas as pl
from jax.experimental.pallas import tpu as pltpu
from jax.experimental.pallas import tpu_sc as plsc

NC = 2
NS = 16
NW = NC * NS
B = 4
NF = 4096
NCRS = 1024
NG = 4096
SLOTS = NW // B

L = 16
QB = 4
U = 4
BIG = 1e30

RT = 2048
CW = 128
NRT = NF // RT
NCW = NG // CW


def _norms(xr, yr, zr, nr, n):
    def body(i, carry):
        off = i * L
        x = xr[pl.ds(off, L)]
        y = yr[pl.ds(off, L)]
        z = zr[pl.ds(off, L)]
        nr[pl.ds(off, L)] = x * x + y * y + z * z
        return carry
    lax.fori_loop(0, n // L, body, 0)


def _fill(ref, n, val):
    v = jnp.full((L,), val, jnp.float32)
    def body(i, carry):
        ref[pl.ds(i * L, L)] = v
        return carry
    lax.fori_loop(0, n // L, body, 0)


def _fused_pass(qx, qy, qz, qn, q0, nq, tx, ty, tz, tn, nt, cm):
    def qgroup(qg, acc):
        qbase = q0 + qg * L
        gqx = qx[pl.ds(qbase, L)]
        gqy = qy[pl.ds(qbase, L)]
        gqz = qz[pl.ds(qbase, L)]
        gqn = qn[pl.ds(qbase, L)]

        for sb in range(L // QB):
            qm2 = []
            qnb = []
            for q in range(QB):
                i = sb * QB + q
                qm2.append((jnp.full((L,), gqx[i]) * -2.0,
                            jnp.full((L,), gqy[i]) * -2.0,
                            jnp.full((L,), gqz[i]) * -2.0))
                qnb.append(jnp.full((L,), gqn[i]))

            def tchunk(tc, carry):
                ms = list(carry)
                base = tc * (U * L)
                for u in range(U):
                    off = base + u * L
                    vx = tx[pl.ds(off, L)]
                    vy = ty[pl.ds(off, L)]
                    vz = tz[pl.ds(off, L)]
                    vn = tn[pl.ds(off, L)]
                    ds = []
                    for q in range(QB):
                        ax, ay, az = qm2[q]
                        d = vn + vx * ax + vy * ay + vz * az
                        ms[q] = jnp.minimum(ms[q], d)
                        ds.append(d + qnb[q])
                    e = jnp.minimum(jnp.minimum(ds[0], ds[1]),
                                    jnp.minimum(ds[2], ds[3]))
                    cm[pl.ds(off, L)] = jnp.minimum(cm[pl.ds(off, L)], e)
                return tuple(ms)

            init = tuple(jnp.full((L,), BIG, jnp.float32)
                         for _ in range(QB))
            ms = lax.fori_loop(0, nt // (U * L), tchunk, init)
            for q in range(QB):
                acc = acc + jnp.min(ms[q]) + gqn[sb * QB + q]
        return acc

    return lax.fori_loop(0, nq // L, qgroup, jnp.float32(0.0))


def _merge_cols(shared, row0, slot, mb, n_per_slot):
    col0 = slot * n_per_slot
    for r in range(SLOTS):
        pltpu.sync_copy(shared.at[row0 + r, pl.ds(col0, n_per_slot)],
                        mb.at[r])

    def chunk(c, acc):
        off = c * L
        m = mb[0, pl.ds(off, L)]
        for r in range(1, SLOTS):
            m = jnp.minimum(m, mb[r, pl.ds(off, L)])
        return acc + m

    sv = lax.fori_loop(0, n_per_slot // L, chunk,
                       jnp.zeros((L,), jnp.float32))
    return jnp.sum(sv)


def _sc_body(gt_hbm, crs_hbm, out_hbm,
             gx, gy, gz, cx, cy, cz, gn, cn,
             cmc, mbc, ob, shc, sem):
    core = lax.axis_index("c")
    sub = lax.axis_index("s")
    wid = core * NS + sub
    b = wid // SLOTS
    slot = wid % SLOTS
    row0 = (sub // SLOTS) * SLOTS

    copies = [
        pltpu.async_copy(gt_hbm.at[b * 3 + 0], gx, sem),
        pltpu.async_copy(gt_hbm.at[b * 3 + 1], gy, sem),
        pltpu.async_copy(gt_hbm.at[b * 3 + 2], gz, sem),
        pltpu.async_copy(crs_hbm.at[b * 3 + 0], cx, sem),
        pltpu.async_copy(crs_hbm.at[b * 3 + 1], cy, sem),
        pltpu.async_copy(crs_hbm.at[b * 3 + 2], cz, sem),
    ]
    for c in copies:
        c.wait()

    _norms(gx, gy, gz, gn, NG)
    _norms(cx, cy, cz, cn, NCRS)
    _fill(cmc, NG, BIG)

    nc_s = NCRS // SLOTS
    s_cg = _fused_pass(cx, cy, cz, cn, slot * nc_s, nc_s,
                       gx, gy, gz, gn, NG, cmc)

    pltpu.sync_copy(cmc, shc.at[sub])
    plsc.subcore_barrier()

    ng_s = NG // SLOTS
    s_gc = _merge_cols(shc, row0, slot, mbc, ng_s)

    lane = lax.iota(jnp.int32, L)
    v = jnp.where(lane == 0, s_cg, jnp.where(lane == 1, s_gc, 0.0))
    ob[...] = v
    pltpu.sync_copy(ob, out_hbm.at[wid])


def _mintree(vals):
    while len(vals) > 1:
        nxt = [jnp.minimum(vals[i], vals[i + 1])
               for i in range(0, len(vals) - 1, 2)]
        if len(vals) % 2:
            nxt.append(vals[-1])
        vals = nxt
    return vals[0]


def _tc_body(x_ref, g_ref, out_ref, cacc):
    b = pl.program_id(0)
    rt = pl.program_id(1)

    xb = x_ref[0]
    gb = g_ref[0]

    @pl.when(rt == 0)
    def _():
        cacc[...] = jnp.full((NCW, 8, CW), BIG, jnp.float32)

    bx2 = jnp.broadcast_to(xb[:, 0:1] * -2.0, (RT, CW))
    by2 = jnp.broadcast_to(xb[:, 1:2] * -2.0, (RT, CW))
    bz2 = jnp.broadcast_to(xb[:, 2:3] * -2.0, (RT, CW))
    xn = jnp.sum(xb * xb, axis=1, keepdims=True)
    bxn = jnp.broadcast_to(xn, (RT, CW))

    rowacc = jnp.full((RT, CW), BIG, jnp.float32)
    for cw in range(NCW):
        c0 = cw * CW
        gx = gb[0:1, c0:c0 + CW]
        gy = gb[1:2, c0:c0 + CW]
        gz = gb[2:3, c0:c0 + CW]
        gn = gx * gx + gy * gy + gz * gz
        d = gn + bx2 * gx + by2 * gy + bz2 * gz
        rowacc = jnp.minimum(rowacc, d)
        e = d + bxn
        cf = _mintree([e[r * 8:(r + 1) * 8, :] for r in range(RT // 8)])
        cacc[cw] = jnp.minimum(cacc[cw], cf)

    s_row = jnp.sum(jnp.min(rowacc, axis=1)) + jnp.sum(xn)
    prev = jnp.where(rt == 0, 0.0, out_ref[b, 0])
    out_ref[b, 0] = prev + s_row

    @pl.when(rt == NRT - 1)
    def _():
        sv = jnp.zeros((1, CW), jnp.float32)
        for cw in range(NCW):
            e = cacc[cw]
            m = _mintree([e[r:r + 1, :] for r in range(8)])
            sv = sv + m
        out_ref[b, 1] = jnp.sum(sv)


def _tc_fine(fine3, gt3):
    return pl.pallas_call(
        _tc_body,
        grid=(B, NRT),
        in_specs=[
            pl.BlockSpec((1, RT, 3), lambda b, rt: (b, rt, 0)),
            pl.BlockSpec((1, 3, NG), lambda b, rt: (b, 0, 0)),
        ],
        out_specs=pl.BlockSpec(memory_space=pltpu.SMEM),
        out_shape=jax.ShapeDtypeStruct((B, 2), jnp.float32),
        scratch_shapes=[
            pltpu.VMEM((NCW, 8, CW), jnp.float32),
        ],
    )(fine3, gt3)


@jax.jit
def kernel(coarse, fine, gt, alpha):
    gt2 = gt.reshape(B * 3, NG)
    fine2 = jnp.transpose(fine, (0, 2, 1)).reshape(B * 3, NF)
    crs2 = jnp.transpose(coarse, (0, 2, 1)).reshape(B * 3, NCRS)

    mesh = plsc.VectorSubcoreMesh(core_axis_name="c", subcore_axis_name="s")
    run = functools.partial(
        pl.kernel,
        mesh=mesh,
        compiler_params=pltpu.CompilerParams(needs_layout_passes=False),
        out_type=jax.ShapeDtypeStruct((NW, L), jnp.float32),
        scratch_types=[
            pltpu.VMEM((NG,), jnp.float32),
            pltpu.VMEM((NG,), jnp.float32),
            pltpu.VMEM((NG,), jnp.float32),
            pltpu.VMEM((NCRS,), jnp.float32),
            pltpu.VMEM((NCRS,), jnp.float32),
            pltpu.VMEM((NCRS,), jnp.float32),
            pltpu.VMEM((NG,), jnp.float32),
            pltpu.VMEM((NCRS,), jnp.float32),
            pltpu.VMEM((NG,), jnp.float32),
            pltpu.VMEM((SLOTS, NG // SLOTS), jnp.float32),
            pltpu.VMEM((L,), jnp.float32),
            pltpu.VMEM_SHARED((NS, NG), jnp.float32),
            pltpu.SemaphoreType.DMA,
        ],
    )(_sc_body)
    sc_part = run(gt2, crs2)
    tc_part = _tc_fine(fine, gt)

    p = sc_part.reshape(B, SLOTS, L)[:, :, :2].sum(axis=1)
    cham_coarse = p[:, 0] / NCRS + p[:, 1] / NG
    cham_fine = tc_part[:, 0] / NF + tc_part[:, 1] / NG
    loss_fine = jnp.mean(cham_fine)
    loss_coarse = jnp.mean(cham_coarse)
    loss = loss_coarse + alpha * loss_fine
    return (loss, loss_coarse, loss_fine)
